# Initial kernel scaffold; baseline (speedup 1.0000x reference)
import jax
import jax.numpy as jnp
from jax.experimental import pallas as pl


def kernel(decoder_hidden, prior_report_emb, prior_report_tokens, ln_g, ln_b, Wq, bq, Wk, bk, Wv, bv, Wo, bo, G1w, G1b, G2w, G2b):
    raise NotImplementedError("write your pallas kernel here")


# speedup vs baseline: 1.3193x; 1.3193x over previous
import functools

import jax
import jax.numpy as jnp
from jax.experimental import pallas as pl

H = 8


def _fused_kernel(dh_ref, emb_ref, ln_g_ref, ln_b_ref,
                  wq_ref, bq_ref, wk_ref, bk_ref, wv_ref, bv_ref,
                  wo_ref, bo_ref, g1w_ref, g1b_ref, g2w_ref, g2b_ref,
                  cc_ref, cp_ref, aw_ref):
    T, D = dh_ref.shape[1], dh_ref.shape[2]
    P = emb_ref.shape[1]
    DH = D // H

    x = dh_ref[0]
    mu = jnp.mean(x, axis=-1, keepdims=True)
    var = jnp.mean((x - mu) ** 2, axis=-1, keepdims=True)
    nh = (x - mu) * jax.lax.rsqrt(var + 1e-5) * ln_g_ref[...] + ln_b_ref[...]

    q = (jnp.dot(nh, wq_ref[...], preferred_element_type=jnp.float32)
         + bq_ref[...]) * (1.0 / jnp.sqrt(jnp.float32(DH)))

    emb = emb_ref[0]
    k = jnp.dot(emb, wk_ref[...], preferred_element_type=jnp.float32) + bk_ref[...]
    v = jnp.dot(emb, wv_ref[...], preferred_element_type=jnp.float32) + bv_ref[...]

    ctx_heads = []
    aw_sum = jnp.zeros((T, P), dtype=jnp.float32)
    for h in range(H):
        q_h = q[:, h * DH:(h + 1) * DH]
        k_h = k[:, h * DH:(h + 1) * DH]
        v_h = v[:, h * DH:(h + 1) * DH]
        s_h = jax.lax.dot_general(
            q_h, k_h, (((1,), (1,)), ((), ())),
            preferred_element_type=jnp.float32)
        m = jnp.max(s_h, axis=-1, keepdims=True)
        e = jnp.exp(s_h - m)
        w_h = e / jnp.sum(e, axis=-1, keepdims=True)
        aw_sum = aw_sum + w_h
        ctx_heads.append(jnp.dot(w_h, v_h, preferred_element_type=jnp.float32))
    aw_ref[0] = aw_sum * (1.0 / H)

    ctx = jnp.concatenate(ctx_heads, axis=1)
    cc = jnp.dot(ctx, wo_ref[...], preferred_element_type=jnp.float32) + bo_ref[...]
    cc_ref[0] = cc

    g = jax.nn.relu(
        jnp.dot(nh, g1w_ref[:D, :], preferred_element_type=jnp.float32)
        + jnp.dot(cc, g1w_ref[D:, :], preferred_element_type=jnp.float32)
        + g1b_ref[...])
    cp_ref[0] = jax.nn.sigmoid(
        jnp.dot(g, g2w_ref[...], preferred_element_type=jnp.float32) + g2b_ref[...])


@functools.partial(jax.jit, static_argnames=())
def kernel(decoder_hidden, prior_report_emb, prior_report_tokens,
           ln_g, ln_b, Wq, bq, Wk, bk, Wv, bv, Wo, bo, G1w, G1b, G2w, G2b):
    B, T, D = decoder_hidden.shape
    P = prior_report_emb.shape[1]

    full = lambda shape: pl.BlockSpec(shape, lambda b: tuple(0 for _ in shape))
    grid_spec = pl.GridSpec(
        grid=(B,),
        in_specs=[
            pl.BlockSpec((1, T, D), lambda b: (b, 0, 0)),
            pl.BlockSpec((1, P, D), lambda b: (b, 0, 0)),
            full(ln_g.shape), full(ln_b.shape),
            full(Wq.shape), full(bq.shape),
            full(Wk.shape), full(bk.shape),
            full(Wv.shape), full(bv.shape),
            full(Wo.shape), full(bo.shape),
            full(G1w.shape), full(G1b.shape),
            full(G2w.shape), full(G2b.shape),
        ],
        out_specs=[
            pl.BlockSpec((1, T, D), lambda b: (b, 0, 0)),
            pl.BlockSpec((1, T, 1), lambda b: (b, 0, 0)),
            pl.BlockSpec((1, T, P), lambda b: (b, 0, 0)),
        ],
    )
    out_shape = [
        jax.ShapeDtypeStruct((B, T, D), jnp.float32),
        jax.ShapeDtypeStruct((B, T, 1), jnp.float32),
        jax.ShapeDtypeStruct((B, T, P), jnp.float32),
    ]
    cc, cp, aw = pl.pallas_call(
        _fused_kernel,
        grid_spec=grid_spec,
        out_shape=out_shape,
    )(decoder_hidden, prior_report_emb, ln_g, ln_b,
      Wq, bq, Wk, bk, Wv, bv, Wo, bo, G1w, G1b, G2w, G2b)
    return (cc, cp, aw)


# speedup vs baseline: 3.5013x; 2.6540x over previous
import jax
import jax.numpy as jnp
from jax.experimental import pallas as pl

H = 8


def _fused_kernel(dh_ref, emb_ref, ln_g_ref, ln_b_ref,
                  wq_ref, bq_ref, wk_ref, bk_ref, wv_ref, bv_ref,
                  wo_ref, bo_ref, g1w_ref, g1b_ref, g2w_ref, g2b_ref,
                  cc_ref, cp_ref, aw_ref):
    T, D = dh_ref.shape[1], dh_ref.shape[2]
    P = emb_ref.shape[1]
    DH = D // H

    x = dh_ref[0]
    mu = jnp.mean(x, axis=-1, keepdims=True)
    var = jnp.mean((x - mu) ** 2, axis=-1, keepdims=True)
    nh = (x - mu) * jax.lax.rsqrt(var + 1e-5) * ln_g_ref[...] + ln_b_ref[...]

    q = (jnp.dot(nh, wq_ref[...], preferred_element_type=jnp.float32)
         + bq_ref[...]) * (1.0 / jnp.sqrt(jnp.float32(DH)))

    wk = wk_ref[...]
    qbk = q * bk_ref[...][None, :]
    a_rows = []
    sb_rows = []
    for h in range(H):
        hs = slice(h * DH, (h + 1) * DH)
        a_rows.append(jax.lax.dot_general(
            q[:, hs], wk[:, hs], (((1,), (1,)), ((), ())),
            preferred_element_type=jnp.float32))
        sb_rows.append(jnp.sum(qbk[:, hs], axis=1, keepdims=True))
    a = jnp.concatenate(a_rows, axis=0)
    sbias = jnp.concatenate(sb_rows, axis=0)

    emb = emb_ref[0]
    scores = jax.lax.dot_general(
        a, emb, (((1,), (1,)), ((), ())),
        preferred_element_type=jnp.float32) + sbias

    m = jnp.max(scores, axis=-1, keepdims=True)
    e = jnp.exp(scores - m)
    w = e / jnp.sum(e, axis=-1, keepdims=True)

    aw = w[:T, :]
    for h in range(1, H):
        aw = aw + w[h * T:(h + 1) * T, :]
    aw_ref[0] = aw * (1.0 / H)

    u = jnp.dot(w, emb, preferred_element_type=jnp.float32)
    wv = wv_ref[...]
    ctx_heads = []
    for h in range(H):
        hs = slice(h * DH, (h + 1) * DH)
        ctx_heads.append(jnp.dot(u[h * T:(h + 1) * T, :], wv[:, hs],
                                 preferred_element_type=jnp.float32))
    ctx = jnp.concatenate(ctx_heads, axis=1) + bv_ref[...][None, :]

    cc = jnp.dot(ctx, wo_ref[...], preferred_element_type=jnp.float32) + bo_ref[...]
    cc_ref[0] = cc

    g = jax.nn.relu(
        jnp.dot(nh, g1w_ref[:D, :], preferred_element_type=jnp.float32)
        + jnp.dot(cc, g1w_ref[D:, :], preferred_element_type=jnp.float32)
        + g1b_ref[...])
    cp_ref[0] = jax.nn.sigmoid(
        jnp.dot(g, g2w_ref[...], preferred_element_type=jnp.float32) + g2b_ref[...])


def kernel(decoder_hidden, prior_report_emb, prior_report_tokens,
           ln_g, ln_b, Wq, bq, Wk, bk, Wv, bv, Wo, bo, G1w, G1b, G2w, G2b):
    B, T, D = decoder_hidden.shape
    P = prior_report_emb.shape[1]

    full = lambda shape: pl.BlockSpec(shape, lambda b: tuple(0 for _ in shape))
    grid_spec = pl.GridSpec(
        grid=(B,),
        in_specs=[
            pl.BlockSpec((1, T, D), lambda b: (b, 0, 0)),
            pl.BlockSpec((1, P, D), lambda b: (b, 0, 0)),
            full(ln_g.shape), full(ln_b.shape),
            full(Wq.shape), full(bq.shape),
            full(Wk.shape), full(bk.shape),
            full(Wv.shape), full(bv.shape),
            full(Wo.shape), full(bo.shape),
            full(G1w.shape), full(G1b.shape),
            full(G2w.shape), full(G2b.shape),
        ],
        out_specs=[
            pl.BlockSpec((1, T, D), lambda b: (b, 0, 0)),
            pl.BlockSpec((1, T, 1), lambda b: (b, 0, 0)),
            pl.BlockSpec((1, T, P), lambda b: (b, 0, 0)),
        ],
    )
    out_shape = [
        jax.ShapeDtypeStruct((B, T, D), jnp.float32),
        jax.ShapeDtypeStruct((B, T, 1), jnp.float32),
        jax.ShapeDtypeStruct((B, T, P), jnp.float32),
    ]
    cc, cp, aw = pl.pallas_call(
        _fused_kernel,
        grid_spec=grid_spec,
        out_shape=out_shape,
    )(decoder_hidden, prior_report_emb, ln_g, ln_b,
      Wq, bq, Wk, bk, Wv, bv, Wo, bo, G1w, G1b, G2w, G2b)
    return (cc, cp, aw)


# speedup vs baseline: 3.5191x; 1.0051x over previous
import jax
import jax.numpy as jnp
from jax.experimental import pallas as pl
from jax.experimental.pallas import tpu as pltpu

H = 8


def _fused_kernel(dh_ref, emb_ref, ln_g_ref, ln_b_ref,
                  wq_ref, bq_ref, wk_ref, bk_ref, wv_ref, bv_ref,
                  wo_ref, bo_ref, g1w_ref, g1b_ref, g2w_ref, g2b_ref,
                  cc_ref, cp_ref, aw_ref):
    T, D = dh_ref.shape[1], dh_ref.shape[2]
    P = emb_ref.shape[1]
    DH = D // H

    x = dh_ref[0]
    mu = jnp.mean(x, axis=-1, keepdims=True)
    var = jnp.mean((x - mu) ** 2, axis=-1, keepdims=True)
    nh = (x - mu) * jax.lax.rsqrt(var + 1e-5) * ln_g_ref[...] + ln_b_ref[...]

    q = (jnp.dot(nh, wq_ref[...], preferred_element_type=jnp.float32)
         + bq_ref[...]) * (1.0 / jnp.sqrt(jnp.float32(DH)))

    wk = wk_ref[...]
    qbk = q * bk_ref[...][None, :]
    a_rows = []
    sb_rows = []
    for h in range(H):
        hs = slice(h * DH, (h + 1) * DH)
        a_rows.append(jax.lax.dot_general(
            q[:, hs], wk[:, hs], (((1,), (1,)), ((), ())),
            preferred_element_type=jnp.float32))
        sb_rows.append(jnp.sum(qbk[:, hs], axis=1, keepdims=True))
    a = jnp.concatenate(a_rows, axis=0)
    sbias = jnp.concatenate(sb_rows, axis=0)

    emb = emb_ref[0]
    scores = jax.lax.dot_general(
        a, emb, (((1,), (1,)), ((), ())),
        preferred_element_type=jnp.float32) + sbias

    m = jnp.max(scores, axis=-1, keepdims=True)
    e = jnp.exp(scores - m)
    w = e / jnp.sum(e, axis=-1, keepdims=True)

    aw = w[:T, :]
    for h in range(1, H):
        aw = aw + w[h * T:(h + 1) * T, :]
    aw_ref[0] = aw * (1.0 / H)

    u = jnp.dot(w, emb, preferred_element_type=jnp.float32)
    wv = wv_ref[...]
    ctx_heads = []
    for h in range(H):
        hs = slice(h * DH, (h + 1) * DH)
        ctx_heads.append(jnp.dot(u[h * T:(h + 1) * T, :], wv[:, hs],
                                 preferred_element_type=jnp.float32))
    ctx = jnp.concatenate(ctx_heads, axis=1) + bv_ref[...][None, :]

    cc = jnp.dot(ctx, wo_ref[...], preferred_element_type=jnp.float32) + bo_ref[...]
    cc_ref[0] = cc

    g = jax.nn.relu(
        jnp.dot(nh, g1w_ref[:D, :], preferred_element_type=jnp.float32)
        + jnp.dot(cc, g1w_ref[D:, :], preferred_element_type=jnp.float32)
        + g1b_ref[...])
    cp_ref[0] = jax.nn.sigmoid(
        jnp.dot(g, g2w_ref[...], preferred_element_type=jnp.float32) + g2b_ref[...])


def kernel(decoder_hidden, prior_report_emb, prior_report_tokens,
           ln_g, ln_b, Wq, bq, Wk, bk, Wv, bv, Wo, bo, G1w, G1b, G2w, G2b):
    B, T, D = decoder_hidden.shape
    P = prior_report_emb.shape[1]

    full = lambda shape: pl.BlockSpec(shape, lambda b: tuple(0 for _ in shape))
    grid_spec = pl.GridSpec(
        grid=(B,),
        in_specs=[
            pl.BlockSpec((1, T, D), lambda b: (b, 0, 0)),
            pl.BlockSpec((1, P, D), lambda b: (b, 0, 0)),
            full(ln_g.shape), full(ln_b.shape),
            full(Wq.shape), full(bq.shape),
            full(Wk.shape), full(bk.shape),
            full(Wv.shape), full(bv.shape),
            full(Wo.shape), full(bo.shape),
            full(G1w.shape), full(G1b.shape),
            full(G2w.shape), full(G2b.shape),
        ],
        out_specs=[
            pl.BlockSpec((1, T, D), lambda b: (b, 0, 0)),
            pl.BlockSpec((1, T, 1), lambda b: (b, 0, 0)),
            pl.BlockSpec((1, T, P), lambda b: (b, 0, 0)),
        ],
    )
    out_shape = [
        jax.ShapeDtypeStruct((B, T, D), jnp.float32),
        jax.ShapeDtypeStruct((B, T, 1), jnp.float32),
        jax.ShapeDtypeStruct((B, T, P), jnp.float32),
    ]
    cc, cp, aw = pl.pallas_call(
        _fused_kernel,
        grid_spec=grid_spec,
        out_shape=out_shape,
        compiler_params=pltpu.CompilerParams(
            dimension_semantics=("parallel",)),
    )(decoder_hidden, prior_report_emb, ln_g, ln_b,
      Wq, bq, Wk, bk, Wv, bv, Wo, bo, G1w, G1b, G2w, G2b)
    return (cc, cp, aw)
